# Initial kernel scaffold; baseline (speedup 1.0000x reference)
import jax
import jax.numpy as jnp
from jax.experimental import pallas as pl


def kernel(features, params):
    raise NotImplementedError("write your pallas kernel here")


# speedup vs baseline: 4.5921x; 4.5921x over previous
import jax
import jax.numpy as jnp
import numpy as np
from jax.experimental import pallas as pl

_B, _N, _F = 1024, 128, 16
_K = 7
_EPS = 1e-3
_HIGHEST = jax.lax.Precision.HIGHEST


def _fold_bn(p):
    g, b, m, v = p
    s = g * jax.lax.rsqrt(v + _EPS)
    return s, b - m * s


def _edge_conv(pts, fts, Wtop, Wbot, b1, W2, b2, W3, b3, Wsc, bsc):
    n = pts.shape[0]
    rA = jnp.sum(pts * pts, axis=1, keepdims=True)
    mm = jax.lax.dot_general(pts, pts, (((1,), (1,)), ((), ())),
                             precision=_HIGHEST)
    d = rA - 2.0 * mm + rA.T
    iota = jax.lax.broadcasted_iota(jnp.int32, (n, n), 1)

    u = jnp.dot(fts, Wbot, precision=_HIGHEST)
    base = jnp.dot(fts, Wtop, precision=_HIGHEST) - u + b1

    acc = jnp.zeros((n, W3.shape[1]), jnp.float32)
    inf = jnp.float32(np.inf)
    for k in range(_K + 1):
        mval = jnp.min(d, axis=1, keepdims=True)
        cand = jnp.where(d == mval, iota, n)
        idx = jnp.min(cand, axis=1, keepdims=True)
        onehot = iota == idx
        d = jnp.where(onehot, inf, d)
        if k == 0:
            continue
        g_k = jnp.dot(onehot.astype(jnp.float32), u,
                      precision=_HIGHEST)
        h = jnp.maximum(base + g_k, 0.0)
        h = jnp.maximum(jnp.dot(h, W2, precision=_HIGHEST) + b2, 0.0)
        h = jnp.maximum(jnp.dot(h, W3, precision=_HIGHEST) + b3, 0.0)
        acc = acc + h
    fts_out = acc * jnp.float32(1.0 / _K)
    sc = jnp.dot(fts, Wsc, precision=_HIGHEST) + bsc
    return jnp.maximum(sc + fts_out, 0.0)


def _body(f_ref, s0_ref, t0_ref,
          wt0_ref, wb0_ref, b10_ref, w20_ref, b20_ref, w30_ref, b30_ref,
          wsc0_ref, bsc0_ref,
          wt1_ref, wb1_ref, b11_ref, w21_ref, b21_ref, w31_ref, b31_ref,
          wsc1_ref, bsc1_ref,
          fcw_ref, fcb_ref, ow_ref, ob_ref, out_ref):
    feats = f_ref[0]
    reduced = jnp.sum(feats, axis=1, keepdims=True)
    mask = (reduced != 0.0).astype(jnp.float32)
    shift = 1e9 * (1.0 - mask)

    eta = feats[:, 0:1] * jnp.cos(feats[:, 1:2])
    phi = feats[:, 0:1] * jnp.sin(feats[:, 1:2])
    points = jnp.concatenate([eta, phi], axis=1)

    fts = feats * s0_ref[...] + t0_ref[...]

    fts = _edge_conv(shift + points, fts,
                     wt0_ref[...], wb0_ref[...], b10_ref[...],
                     w20_ref[...], b20_ref[...], w30_ref[...], b30_ref[...],
                     wsc0_ref[...], bsc0_ref[...])
    fts = _edge_conv(shift + fts, fts,
                     wt1_ref[...], wb1_ref[...], b11_ref[...],
                     w21_ref[...], b21_ref[...], w31_ref[...], b31_ref[...],
                     wsc1_ref[...], bsc1_ref[...])

    fts = fts * mask
    pool = jnp.sum(fts, axis=0, keepdims=True) * jnp.float32(1.0 / _N)
    x = jnp.maximum(jnp.dot(pool, fcw_ref[...], precision=_HIGHEST)
                    + fcb_ref[...], 0.0)
    logits = jnp.dot(x, ow_ref[...], precision=_HIGHEST) + ob_ref[...]
    z = logits - jnp.max(logits, axis=1, keepdims=True)
    e = jnp.exp(z)
    out_ref[0] = e / jnp.sum(e, axis=1, keepdims=True)


def _prep_weights(params):
    s0, t0 = _fold_bn(params["bn0"])
    ws = [s0.reshape(1, -1), t0.reshape(1, -1)]
    for layer in params["layers"]:
        w1, w2, w3 = layer["ws"]
        c_in = w1.shape[0] // 2
        s1, t1 = _fold_bn(layer["bns"][0])
        s2, t2 = _fold_bn(layer["bns"][1])
        s3, t3 = _fold_bn(layer["bns"][2])
        ssc, tsc = _fold_bn(layer["bnsc"])
        ws += [w1[:c_in] * s1, w1[c_in:] * s1, t1.reshape(1, -1),
               w2 * s2, t2.reshape(1, -1),
               w3 * s3, t3.reshape(1, -1),
               layer["wsc"] * ssc, tsc.reshape(1, -1)]
    ws += [params["fc_w"], params["fc_b"].reshape(1, -1),
           params["out_w"], params["out_b"].reshape(1, -1)]
    return ws


def kernel(features, params):
    ws = _prep_weights(params)
    full = lambda a: pl.BlockSpec(a.shape, lambda i: (0,) * a.ndim)
    out = pl.pallas_call(
        _body,
        grid=(_B,),
        in_specs=[pl.BlockSpec((1, _N, _F), lambda i: (i, 0, 0))]
                 + [full(a) for a in ws],
        out_specs=pl.BlockSpec((1, 1, 5), lambda i: (i, 0, 0)),
        out_shape=jax.ShapeDtypeStruct((_B, 1, 5), jnp.float32),
    )(features, *ws)
    return out.reshape(_B, 5)
